# baseline (speedup 1.0000x reference)
import functools

import jax
import jax.numpy as jnp
from jax import lax
from jax.experimental import pallas as pl
from jax.experimental.pallas import tpu as pltpu
from jax.experimental.pallas import tpu_sc as plsc

_NC = 2
_NS = 16
_NW = _NC * _NS
_LANES = 16
_CHUNK = 128


@functools.partial(jax.jit, static_argnames=("n_tok", "dim"))
def _sc_embed(ids, voc, pos, lenx, llm_table, cod_table, *, n_tok, dim):
    per_w = n_tok // _NW
    n_chunks = per_w // _CHUNK

    def body(ids_hbm, voc_hbm, pos_hbm, lenx_hbm, llm_hbm, cod_hbm,
             out_hbm, mask_hbm,
             ids_v, voc_v, pos_v, lenx_v, illm_v, icod_v, maskf_v, mi_v,
             llm_rows, cod_rows, sem1, sem2):
        wid = lax.axis_index("s") * _NC + lax.axis_index("c")
        base0 = wid * per_w

        def chunk(c, carry):
            base = base0 + c * _CHUNK
            dsl = pl.ds(base, _CHUNK)
            pltpu.sync_copy(ids_hbm.at[dsl], ids_v)
            pltpu.sync_copy(voc_hbm.at[dsl], voc_v)
            pltpu.sync_copy(pos_hbm.at[dsl], pos_v)
            pltpu.sync_copy(lenx_hbm.at[dsl], lenx_v)

            def idx_body(j, carry2):
                sl = pl.ds(j * _LANES, _LANES)
                idv = ids_v[sl]
                vv = voc_v[sl]
                mi = lax.shift_right_logical(pos_v[sl] - lenx_v[sl], 31)
                sel = mi * idv
                illm_v[sl] = sel * (1 - vv)
                icod_v[sl] = sel * vv
                maskf_v[sl] = mi.astype(jnp.float32)
                mi_v[sl] = mi
                return carry2

            lax.fori_loop(0, _CHUNK // _LANES, idx_body, 0)

            cp1 = pltpu.async_copy(llm_hbm.at[illm_v], llm_rows, sem1)
            cp2 = pltpu.async_copy(cod_hbm.at[icod_v], cod_rows, sem2)
            cp1.wait()
            cp2.wait()

            def comb(i, carry2):
                mvec = plsc.load_gather(
                    maskf_v, [jnp.zeros((_LANES,), jnp.int32) + i])
                for d in range(dim // _LANES):
                    sl = pl.ds(d * _LANES, _LANES)
                    llm_rows[i, sl] = (llm_rows[i, sl] + cod_rows[i, sl]) * mvec
                return carry2

            lax.fori_loop(0, _CHUNK, comb, 0)


            pltpu.sync_copy(llm_rows, out_hbm.at[dsl])
            pltpu.sync_copy(mi_v, mask_hbm.at[dsl])
            return carry

        lax.fori_loop(0, n_chunks, chunk, 0)

    fn = pl.kernel(
        body,
        out_type=[
            jax.ShapeDtypeStruct((n_tok, dim), jnp.float32),
            jax.ShapeDtypeStruct((n_tok,), jnp.int32),
        ],
        mesh=plsc.VectorSubcoreMesh(core_axis_name="c", subcore_axis_name="s"),
        compiler_params=pltpu.CompilerParams(
            use_tc_tiling_on_sc=False, needs_layout_passes=False),
        scratch_types=[
            pltpu.VMEM((_CHUNK,), jnp.int32),
            pltpu.VMEM((_CHUNK,), jnp.int32),
            pltpu.VMEM((_CHUNK,), jnp.int32),
            pltpu.VMEM((_CHUNK,), jnp.int32),
            pltpu.VMEM((_CHUNK,), jnp.int32),
            pltpu.VMEM((_CHUNK,), jnp.int32),
            pltpu.VMEM((_CHUNK,), jnp.float32),
            pltpu.VMEM((_CHUNK,), jnp.int32),
            pltpu.VMEM((_CHUNK, dim), jnp.float32),
            pltpu.VMEM((_CHUNK, dim), jnp.float32),
            pltpu.SemaphoreType.DMA,
            pltpu.SemaphoreType.DMA,
        ],
    )
    return fn(ids, voc, pos, lenx, llm_table, cod_table)


def kernel(input_ids, vocab_ids, length, llm_table, cod_table):
    B, L = input_ids.shape
    _, D = llm_table.shape
    N = B * L
    ids = input_ids.reshape(N).astype(jnp.int32)
    voc = vocab_ids.reshape(N).astype(jnp.int32)
    pos = jnp.tile(lax.iota(jnp.int32, L), B)
    lenx = jnp.broadcast_to(
        length.astype(jnp.int32)[:, None], (B, L)).reshape(N)
    out, mask_i = _sc_embed(ids, voc, pos, lenx, llm_table, cod_table,
                            n_tok=N, dim=D)
    return out.reshape(B, L, D), (mask_i.reshape(B, L) != 0)
